# baseline (speedup 1.0000x reference)
import functools

import jax
import jax.numpy as jnp
from jax import lax
from jax.experimental import pallas as pl
from jax.experimental.pallas import tpu as pltpu
from jax.experimental.pallas import tpu_sc as plsc

_N = 10000
_E = 320000
_H = 128
_F = 4

_NC = 2
_NS = 16
_NW = _NC * _NS
_EPW = _E // _NW
_K = 128
_NBK = 80
_EP = _NW * _NBK * _K
_CH = 8
_ZR = 8
_NPW = 1280


def _enc_body(ea_ref, dst_ref, enc_ref):
    a = ea_ref[...]
    s = jnp.sum(a * a, axis=0)
    w = jnp.exp(-4.0 * jnp.sqrt(s + 1e-12))
    dm8 = lax.convert_element_type(
        lax.bitwise_and(dst_ref[...], jnp.int32(7)), jnp.float32)
    enc_ref[...] = dm8 + w * 0.0625


def _edge_enc(edge_attr, dst):
    rows = _E // 128
    ea = edge_attr.T.reshape(_F, rows, 128)
    enc = pl.pallas_call(
        _enc_body,
        out_shape=jax.ShapeDtypeStruct((rows, 128), jnp.float32),
    )(ea, dst.reshape(rows, 128))
    return enc.reshape(_E)


def _sc_agg_body(src_hbm, dst_hbm, enc_hbm, x_hbm, out_x, out_w,
                 acc, acw, src_v, dst_v, enc_v, rows_v, wrow_v, zrow_v, sem):
    cid = lax.axis_index("c")
    sid = lax.axis_index("s")
    wid = sid * _NC + cid
    zero16 = jnp.zeros((16,), jnp.float32)
    one16 = jnp.full((16,), 1.0, jnp.float32)

    def _zz(i, c):
        for j in range(_H // 16):
            zrow_v[i, pl.ds(j * 16, 16)] = zero16
        return c
    lax.fori_loop(0, _ZR, _zz, 0)

    @pl.when(sid < 10)
    def _zx():
        def _zb(r, c):
            pltpu.sync_copy(zrow_v, acc.at[pl.ds(sid * 1000 + r * _ZR, _ZR)])
            return c
        lax.fori_loop(0, 1000 // _ZR, _zb, 0)

    def _zw(r, c):
        pltpu.sync_copy(zrow_v, acw.at[pl.ds(sid * (_NPW // _NS) + r * _ZR, _ZR)])
        return c
    lax.fori_loop(0, _NPW // _NS // _ZR, _zw, 0)
    plsc.subcore_barrier()

    def _chunk(ci, cc0):
        pltpu.sync_copy(src_hbm.at[wid, pl.ds(ci * _CH, _CH)], src_v)
        pltpu.sync_copy(dst_hbm.at[wid, pl.ds(ci * _CH, _CH)], dst_v)
        pltpu.sync_copy(enc_hbm.at[wid, pl.ds(ci * _CH, _CH)], enc_v)

        def _batch(b, c):
            pltpu.async_copy(x_hbm.at[src_v.at[b]], rows_v, sem).wait()

            def _scale(g, cc):
                evec = enc_v[b, pl.ds(g * 16, 16)]
                dvec = dst_v[b, pl.ds(g * 16, 16)]
                for i in range(16):
                    e = g * 16 + i
                    gi = jnp.full((16, 1), i, jnp.int32)
                    dn = lax.GatherDimensionNumbers(offset_dims=(),
                                                    collapsed_slice_dims=(0,),
                                                    start_index_map=(0,))
                    eb = lax.gather(evec, gi, dn, (1,),
                                    mode=lax.GatherScatterMode.PROMISE_IN_BOUNDS)
                    fr = lax.rem(eb, one16)
                    wb = fr * 16.0
                    dmf = eb - fr
                    for j in range(_H // 16):
                        sl = pl.ds(j * 16, 16)
                        rows_v[e, sl] = rows_v[e, sl] * wb
                        wrow_v[e, sl] = wrow_v[e, sl] * 0.0 + jnp.where(
                            dmf == jnp.full((16,), float(j), jnp.float32),
                            wb, zero16)
                idx2 = lax.shift_right_logical(dvec, 3)
                pltpu.sync_copy(wrow_v.at[pl.ds(g * 16, 16)], acw.at[idx2],
                                add=True)
                return cc
            lax.fori_loop(0, _K // 16, _scale, 0)

            pltpu.sync_copy(rows_v, acc.at[dst_v.at[b]], add=True)
            return c
        lax.fori_loop(0, _CH, _batch, 0)
        return cc0
    lax.fori_loop(0, _NBK // _CH, _chunk, 0)
    plsc.subcore_barrier()

    @pl.when(sid < 10)
    def _copy_out():
        r0 = sid * 1000
        pltpu.sync_copy(acc.at[pl.ds(r0, 1000)], out_x.at[cid, pl.ds(r0, 1000)])
        r1 = sid * (_NPW // 10)
        pltpu.sync_copy(acw.at[pl.ds(r1, _NPW // 10)],
                        out_w.at[cid, pl.ds(r1, _NPW // 10)])


def _sc_aggregate(src, dst, enc, x):
    pad = _EP - _E
    zi = jnp.zeros((pad,), jnp.int32)
    src = jnp.concatenate([src, zi]).reshape(_NW, _NBK, _K)
    dst = jnp.concatenate([dst, zi]).reshape(_NW, _NBK, _K)
    enc = jnp.concatenate([enc, jnp.zeros((pad,), jnp.float32)]).reshape(
        _NW, _NBK, _K)
    mesh = plsc.VectorSubcoreMesh(core_axis_name="c", subcore_axis_name="s")
    f = pl.kernel(
        _sc_agg_body,
        out_type=(jax.ShapeDtypeStruct((_NC, _N, _H), jnp.float32),
                  jax.ShapeDtypeStruct((_NC, _NPW, _H), jnp.float32)),
        mesh=mesh,
        scratch_types=[
            pltpu.VMEM_SHARED((_N, _H), jnp.float32),
            pltpu.VMEM_SHARED((_NPW, _H), jnp.float32),
            pltpu.VMEM((_CH, _K), jnp.int32),
            pltpu.VMEM((_CH, _K), jnp.int32),
            pltpu.VMEM((_CH, _K), jnp.float32),
            pltpu.VMEM((_K, _H), jnp.float32),
            pltpu.VMEM((_K, _H), jnp.float32),
            pltpu.VMEM((_ZR, _H), jnp.float32),
            pltpu.SemaphoreType.DMA,
        ],
    )
    return f(src, dst, enc, x)


def _mlp_body(px_ref, pw_ref, x_ref, w1_ref, b1_ref, w2_ref, b2_ref,
              g_ref, bt_ref, o_ref):
    agg = px_ref[0] + px_ref[1]
    wp = pw_ref[0] + pw_ref[1]
    ws = wp[:_N // 8].reshape(_N // 8, 8, 16)[:, :, 0].reshape(_N, 1)
    agg = agg / jnp.maximum(ws, 1e-6)
    h = jnp.dot(agg, w1_ref[...], preferred_element_type=jnp.float32) + b1_ref[...]
    h = 0.5 * h * (1.0 + lax.erf(h * (2.0 ** -0.5)))
    msg = jnp.dot(h, w2_ref[...], preferred_element_type=jnp.float32) + b2_ref[...]
    y = x_ref[...] + msg
    mu = jnp.mean(y, axis=-1, keepdims=True)
    yc = y - mu
    var = jnp.mean(yc * yc, axis=-1, keepdims=True)
    o_ref[...] = yc * lax.rsqrt(var + 1e-5) * g_ref[...] + bt_ref[...]


def _mlp(px, pw, x, W1, b1, W2, b2, gamma, beta):
    vec = lambda v: v.reshape(1, _H)
    return pl.pallas_call(
        _mlp_body,
        out_shape=jax.ShapeDtypeStruct((_N, _H), jnp.float32),
    )(px, pw, x, W1, vec(b1), W2, vec(b2), vec(gamma), vec(beta))


def kernel(x, edge_index, edge_attr, W1, b1, W2, b2, gamma, beta):
    src = edge_index[0]
    dst = edge_index[1]
    enc = _edge_enc(edge_attr, dst)
    px, pw = _sc_aggregate(src, dst, enc, x)
    return _mlp(px, pw, x, W1, b1, W2, b2, gamma, beta)


# speedup vs baseline: 1.8148x; 1.8148x over previous
import functools

import jax
import jax.numpy as jnp
from jax import lax
from jax.experimental import pallas as pl
from jax.experimental.pallas import tpu as pltpu
from jax.experimental.pallas import tpu_sc as plsc

_N = 10000
_E = 320000
_H = 128
_F = 4

_NC = 2
_NS = 16
_NW = _NC * _NS
_EPW = _E // _NW
_K = 80
_NB = _EPW // _K
_ZR = 8
_NPW = 1280


def _enc_body(ea_ref, dst_ref, enc_ref):
    a = ea_ref[...]
    s = jnp.sum(a * a, axis=0)
    w = jnp.exp(-4.0 * jnp.sqrt(s + 1e-12))
    dm8 = lax.convert_element_type(
        lax.bitwise_and(dst_ref[...], jnp.int32(7)), jnp.float32)
    enc_ref[...] = dm8 + w * 0.0625


def _edge_enc(edge_attr, dst):
    rows = _E // 128
    ea = edge_attr.T.reshape(_F, rows, 128)
    enc = pl.pallas_call(
        _enc_body,
        out_shape=jax.ShapeDtypeStruct((rows, 128), jnp.float32),
    )(ea, dst.reshape(rows, 128))
    return enc.reshape(_E)


def _sc_agg_body(src_hbm, dst_hbm, enc_hbm, x_hbm, out_x, out_w,
                 acc, acw, src_v, dst_v, enc_v, rows_v, wrow_v, zrow_v,
                 sem, sem_i, sem_s, sem_w):
    cid = lax.axis_index("c")
    sid = lax.axis_index("s")
    wid = sid * _NC + cid
    zero16 = jnp.zeros((16,), jnp.float32)
    one16 = jnp.full((16,), 1.0, jnp.float32)

    def _zz(i, c):
        for j in range(_H // 16):
            zrow_v[i, pl.ds(j * 16, 16)] = zero16
        return c
    lax.fori_loop(0, _ZR, _zz, 0)

    @pl.when(sid < 10)
    def _zx():
        def _zb(r, c):
            pltpu.sync_copy(zrow_v, acc.at[pl.ds(sid * 1000 + r * _ZR, _ZR)])
            return c
        lax.fori_loop(0, 1000 // _ZR, _zb, 0)

    def _zw(r, c):
        pltpu.sync_copy(zrow_v, acw.at[pl.ds(sid * (_NPW // _NS) + r * _ZR, _ZR)])
        return c
    lax.fori_loop(0, _NPW // _NS // _ZR, _zw, 0)
    plsc.subcore_barrier()

    def _batch(b, c):
        @pl.when(b > 0)
        def _drain():
            pltpu.make_async_copy(rows_v, acc.at[dst_v], sem_s).wait()
            for g in range(_K // 16):
                pltpu.make_async_copy(wrow_v.at[pl.ds(g * 16, 16)],
                                      acw.at[dst_v.at[pl.ds(g * 16, 16)]],
                                      sem_w).wait()
        base = wid * _EPW + b * _K
        c1 = pltpu.async_copy(src_hbm.at[pl.ds(base, _K)], src_v, sem_i)
        c2 = pltpu.async_copy(dst_hbm.at[pl.ds(base, _K)], dst_v, sem_i)
        c3 = pltpu.async_copy(enc_hbm.at[pl.ds(base, _K)], enc_v, sem_i)
        c1.wait(); c2.wait(); c3.wait()
        pltpu.async_copy(x_hbm.at[src_v], rows_v, sem).wait()

        def _scale(g, cc):
            evec = enc_v[pl.ds(g * 16, 16)]
            dvec = dst_v[pl.ds(g * 16, 16)]
            for i in range(16):
                e = g * 16 + i
                gi = jnp.full((16, 1), i, jnp.int32)
                dn = lax.GatherDimensionNumbers(offset_dims=(),
                                                collapsed_slice_dims=(0,),
                                                start_index_map=(0,))
                eb = lax.gather(evec, gi, dn, (1,),
                                mode=lax.GatherScatterMode.PROMISE_IN_BOUNDS)
                fr = lax.rem(eb, one16)
                wb = fr * 16.0
                dmf = eb - fr
                for j in range(_H // 16):
                    sl = pl.ds(j * 16, 16)
                    rows_v[e, sl] = rows_v[e, sl] * wb
                    wrow_v[e, sl] = wrow_v[e, sl] * 0.0 + jnp.where(
                        dmf == jnp.full((16,), float(j), jnp.float32),
                        wb, zero16)
            idx2 = lax.shift_right_logical(dvec, 3)
            pltpu.async_copy(wrow_v.at[pl.ds(g * 16, 16)], acw.at[idx2],
                             sem_w, add=True)
            return cc
        lax.fori_loop(0, _K // 16, _scale, 0)

        pltpu.async_copy(rows_v, acc.at[dst_v], sem_s, add=True)
        return c
    lax.fori_loop(0, _NB, _batch, 0)
    pltpu.make_async_copy(rows_v, acc.at[dst_v], sem_s).wait()
    for g in range(_K // 16):
        pltpu.make_async_copy(wrow_v.at[pl.ds(g * 16, 16)],
                              acw.at[dst_v.at[pl.ds(g * 16, 16)]],
                              sem_w).wait()
    plsc.subcore_barrier()

    @pl.when(sid < 10)
    def _copy_out():
        r0 = sid * 1000
        pltpu.sync_copy(acc.at[pl.ds(r0, 1000)], out_x.at[cid, pl.ds(r0, 1000)])
        r1 = sid * (_NPW // 10)
        pltpu.sync_copy(acw.at[pl.ds(r1, _NPW // 10)],
                        out_w.at[cid, pl.ds(r1, _NPW // 10)])


def _sc_aggregate(src, dst, enc, x):
    mesh = plsc.VectorSubcoreMesh(core_axis_name="c", subcore_axis_name="s")
    f = pl.kernel(
        _sc_agg_body,
        out_type=(jax.ShapeDtypeStruct((_NC, _N, _H), jnp.float32),
                  jax.ShapeDtypeStruct((_NC, _NPW, _H), jnp.float32)),
        mesh=mesh,
        scratch_types=[
            pltpu.VMEM_SHARED((_N, _H), jnp.float32),
            pltpu.VMEM_SHARED((_NPW, _H), jnp.float32),
            pltpu.VMEM((_K,), jnp.int32),
            pltpu.VMEM((_K,), jnp.int32),
            pltpu.VMEM((_K,), jnp.float32),
            pltpu.VMEM((_K, _H), jnp.float32),
            pltpu.VMEM((_K, _H), jnp.float32),
            pltpu.VMEM((_ZR, _H), jnp.float32),
            pltpu.SemaphoreType.DMA,
            pltpu.SemaphoreType.DMA,
            pltpu.SemaphoreType.DMA,
            pltpu.SemaphoreType.DMA,
        ],
    )
    return f(src, dst, enc, x)


def _mlp_body(px_ref, pw_ref, x_ref, w1_ref, b1_ref, w2_ref, b2_ref,
              g_ref, bt_ref, o_ref):
    agg = px_ref[0] + px_ref[1]
    wp = pw_ref[0] + pw_ref[1]
    ws = wp[:_N // 8].reshape(_N // 8, 8, 16)[:, :, 0].reshape(_N, 1)
    agg = agg / jnp.maximum(ws, 1e-6)
    h = jnp.dot(agg, w1_ref[...], preferred_element_type=jnp.float32) + b1_ref[...]
    h = 0.5 * h * (1.0 + lax.erf(h * (2.0 ** -0.5)))
    msg = jnp.dot(h, w2_ref[...], preferred_element_type=jnp.float32) + b2_ref[...]
    y = x_ref[...] + msg
    mu = jnp.mean(y, axis=-1, keepdims=True)
    yc = y - mu
    var = jnp.mean(yc * yc, axis=-1, keepdims=True)
    o_ref[...] = yc * lax.rsqrt(var + 1e-5) * g_ref[...] + bt_ref[...]


def _mlp(px, pw, x, W1, b1, W2, b2, gamma, beta):
    vec = lambda v: v.reshape(1, _H)
    return pl.pallas_call(
        _mlp_body,
        out_shape=jax.ShapeDtypeStruct((_N, _H), jnp.float32),
    )(px, pw, x, W1, vec(b1), W2, vec(b2), vec(gamma), vec(beta))


def kernel(x, edge_index, edge_attr, W1, b1, W2, b2, gamma, beta):
    src = edge_index[0]
    dst = edge_index[1]
    enc = _edge_enc(edge_attr, dst)
    px, pw = _sc_aggregate(src, dst, enc, x)
    return _mlp(px, pw, x, W1, b1, W2, b2, gamma, beta)


# speedup vs baseline: 1.8624x; 1.0262x over previous
import functools

import jax
import jax.numpy as jnp
from jax import lax
from jax.experimental import pallas as pl
from jax.experimental.pallas import tpu as pltpu
from jax.experimental.pallas import tpu_sc as plsc

_N = 10000
_E = 320000
_H = 128
_F = 4

_NC = 2
_NS = 16
_NW = _NC * _NS
_EPW = _E // _NW
_K = 80
_NB = _EPW // _K
_ZR = 8
_NPW = 1280


def _enc_body(ea_ref, dst_ref, enc_ref):
    a = ea_ref[...]
    s = jnp.sum(a * a, axis=0)
    w = jnp.exp(-4.0 * jnp.sqrt(s + 1e-12))
    dm8 = lax.convert_element_type(
        lax.bitwise_and(dst_ref[...], jnp.int32(7)), jnp.float32)
    enc_ref[...] = dm8 + w * 0.0625


def _edge_enc(edge_attr, dst):
    rows = _E // 128
    ea = edge_attr.T.reshape(_F, rows, 128)
    enc = pl.pallas_call(
        _enc_body,
        out_shape=jax.ShapeDtypeStruct((rows, 128), jnp.float32),
    )(ea, dst.reshape(rows, 128))
    return enc.reshape(_E)


def _sc_agg_body(src_hbm, dst_hbm, enc_hbm, x_hbm, out_x, out_w,
                 acc, acw, src_v, dst_v, enc_v, rows_v, wrow_v, zrow_v,
                 sem, sem_i, sem_s, sem_w, sem_g2):
    cid = lax.axis_index("c")
    sid = lax.axis_index("s")
    wid = sid * _NC + cid
    zero16 = jnp.zeros((16,), jnp.float32)
    one16 = jnp.full((16,), 1.0, jnp.float32)

    def _zz(i, c):
        for j in range(_H // 16):
            zrow_v[i, pl.ds(j * 16, 16)] = zero16
        return c
    lax.fori_loop(0, _ZR, _zz, 0)

    @pl.when(sid < 10)
    def _zx():
        def _zb(r, c):
            pltpu.sync_copy(zrow_v, acc.at[pl.ds(sid * 1000 + r * _ZR, _ZR)])
            return c
        lax.fori_loop(0, 1000 // _ZR, _zb, 0)

    def _zw(r, c):
        pltpu.sync_copy(zrow_v, acw.at[pl.ds(sid * (_NPW // _NS) + r * _ZR, _ZR)])
        return c
    lax.fori_loop(0, _NPW // _NS // _ZR, _zw, 0)
    plsc.subcore_barrier()

    def _batch(b, c):
        @pl.when(b > 0)
        def _drain():
            pltpu.make_async_copy(rows_v, acc.at[dst_v], sem_s).wait()
            for g in range(_K // 16):
                pltpu.make_async_copy(wrow_v.at[pl.ds(g * 16, 16)],
                                      acw.at[dst_v.at[pl.ds(g * 16, 16)]],
                                      sem_w).wait()
        base = wid * _EPW + b * _K
        c1 = pltpu.async_copy(src_hbm.at[pl.ds(base, _K)], src_v, sem_i)
        c2 = pltpu.async_copy(dst_hbm.at[pl.ds(base, _K)], dst_v, sem_i)
        c3 = pltpu.async_copy(enc_hbm.at[pl.ds(base, _K)], enc_v, sem_i)
        c1.wait(); c2.wait(); c3.wait()
        g1 = pltpu.async_copy(x_hbm.at[src_v.at[pl.ds(0, 48)]],
                              rows_v.at[pl.ds(0, 48)], sem)
        g2 = pltpu.async_copy(x_hbm.at[src_v.at[pl.ds(48, 32)]],
                              rows_v.at[pl.ds(48, 32)], sem_g2)

        def _scale(g, cc):
            evec = enc_v[pl.ds(g * 16, 16)]
            dvec = dst_v[pl.ds(g * 16, 16)]
            for i in range(16):
                e = g * 16 + i
                gi = jnp.full((16, 1), i, jnp.int32)
                dn = lax.GatherDimensionNumbers(offset_dims=(),
                                                collapsed_slice_dims=(0,),
                                                start_index_map=(0,))
                eb = lax.gather(evec, gi, dn, (1,),
                                mode=lax.GatherScatterMode.PROMISE_IN_BOUNDS)
                fr = lax.rem(eb, one16)
                wb = fr * 16.0
                dmf = eb - fr
                for j in range(_H // 16):
                    sl = pl.ds(j * 16, 16)
                    rows_v[e, sl] = rows_v[e, sl] * wb
                    wrow_v[e, sl] = wrow_v[e, sl] * 0.0 + jnp.where(
                        dmf == jnp.full((16,), float(j), jnp.float32),
                        wb, zero16)
            idx2 = lax.shift_right_logical(dvec, 3)
            pltpu.async_copy(wrow_v.at[pl.ds(g * 16, 16)], acw.at[idx2],
                             sem_w, add=True)
            return cc
        g1.wait()
        lax.fori_loop(0, 3, _scale, 0)
        g2.wait()
        lax.fori_loop(3, _K // 16, _scale, 0)

        pltpu.async_copy(rows_v, acc.at[dst_v], sem_s, add=True)
        return c
    lax.fori_loop(0, _NB, _batch, 0)
    pltpu.make_async_copy(rows_v, acc.at[dst_v], sem_s).wait()
    for g in range(_K // 16):
        pltpu.make_async_copy(wrow_v.at[pl.ds(g * 16, 16)],
                              acw.at[dst_v.at[pl.ds(g * 16, 16)]],
                              sem_w).wait()
    plsc.subcore_barrier()

    @pl.when(sid < 10)
    def _copy_out():
        r0 = sid * 1000
        pltpu.sync_copy(acc.at[pl.ds(r0, 1000)], out_x.at[cid, pl.ds(r0, 1000)])
        r1 = sid * (_NPW // 10)
        pltpu.sync_copy(acw.at[pl.ds(r1, _NPW // 10)],
                        out_w.at[cid, pl.ds(r1, _NPW // 10)])


def _sc_aggregate(src, dst, enc, x):
    mesh = plsc.VectorSubcoreMesh(core_axis_name="c", subcore_axis_name="s")
    f = pl.kernel(
        _sc_agg_body,
        out_type=(jax.ShapeDtypeStruct((_NC, _N, _H), jnp.float32),
                  jax.ShapeDtypeStruct((_NC, _NPW, _H), jnp.float32)),
        mesh=mesh,
        scratch_types=[
            pltpu.VMEM_SHARED((_N, _H), jnp.float32),
            pltpu.VMEM_SHARED((_NPW, _H), jnp.float32),
            pltpu.VMEM((_K,), jnp.int32),
            pltpu.VMEM((_K,), jnp.int32),
            pltpu.VMEM((_K,), jnp.float32),
            pltpu.VMEM((_K, _H), jnp.float32),
            pltpu.VMEM((_K, _H), jnp.float32),
            pltpu.VMEM((_ZR, _H), jnp.float32),
            pltpu.SemaphoreType.DMA,
            pltpu.SemaphoreType.DMA,
            pltpu.SemaphoreType.DMA,
            pltpu.SemaphoreType.DMA,
            pltpu.SemaphoreType.DMA,
        ],
    )
    return f(src, dst, enc, x)


def _mlp_body(px_ref, pw_ref, x_ref, w1_ref, b1_ref, w2_ref, b2_ref,
              g_ref, bt_ref, o_ref):
    agg = px_ref[0] + px_ref[1]
    wp = pw_ref[0] + pw_ref[1]
    ws = wp[:_N // 8].reshape(_N // 8, 8, 16)[:, :, 0].reshape(_N, 1)
    agg = agg / jnp.maximum(ws, 1e-6)
    h = jnp.dot(agg, w1_ref[...], preferred_element_type=jnp.float32) + b1_ref[...]
    h = 0.5 * h * (1.0 + lax.erf(h * (2.0 ** -0.5)))
    msg = jnp.dot(h, w2_ref[...], preferred_element_type=jnp.float32) + b2_ref[...]
    y = x_ref[...] + msg
    mu = jnp.mean(y, axis=-1, keepdims=True)
    yc = y - mu
    var = jnp.mean(yc * yc, axis=-1, keepdims=True)
    o_ref[...] = yc * lax.rsqrt(var + 1e-5) * g_ref[...] + bt_ref[...]


def _mlp(px, pw, x, W1, b1, W2, b2, gamma, beta):
    vec = lambda v: v.reshape(1, _H)
    return pl.pallas_call(
        _mlp_body,
        out_shape=jax.ShapeDtypeStruct((_N, _H), jnp.float32),
    )(px, pw, x, W1, vec(b1), W2, vec(b2), vec(gamma), vec(beta))


def kernel(x, edge_index, edge_attr, W1, b1, W2, b2, gamma, beta):
    src = edge_index[0]
    dst = edge_index[1]
    enc = _edge_enc(edge_attr, dst)
    px, pw = _sc_aggregate(src, dst, enc, x)
    return _mlp(px, pw, x, W1, b1, W2, b2, gamma, beta)
